# initial kernel scaffold (unmeasured)
import jax
import jax.numpy as jnp
from jax import lax
from jax.experimental import pallas as pl
from jax.experimental.pallas import tpu as pltpu

N_DEV = 8


def kernel(x, w_mat):
    m_per, k = x.shape
    n = w_mat.shape[1]
    n_per = n // N_DEV
    m_total = N_DEV * m_per

    def body(x_ref, w_ref, out_ref, send_buf, recv_buf, send_sems, recv_sems):
        my = lax.axis_index("i")

        barrier = pltpu.get_barrier_semaphore()
        for j in range(N_DEV):
            @pl.when(my != j)
            def _():
                pl.semaphore_signal(
                    barrier, inc=1, device_id=(j,),
                    device_id_type=pl.DeviceIdType.MESH)
        pl.semaphore_wait(barrier, N_DEV - 1)

        y = jnp.dot(x_ref[:, :], w_ref[:, :],
                    preferred_element_type=jnp.float32)
        c = 0.7978845608028654
        y = 0.5 * y * (1.0 + jnp.tanh(c * (y + 0.044715 * y * y * y)))
        yb = y.astype(jnp.bfloat16)
        for j in range(N_DEV):
            send_buf[j, :, :] = yb[:, j * n_per:(j + 1) * n_per]

        for j in range(N_DEV):
            @pl.when(my != j)
            def _():
                pltpu.make_async_remote_copy(
                    src_ref=send_buf.at[j],
                    dst_ref=recv_buf.at[my],
                    send_sem=send_sems.at[j],
                    recv_sem=recv_sems.at[my],
                    device_id=(j,),
                    device_id_type=pl.DeviceIdType.MESH,
                ).start()

        own = lax.dynamic_slice(yb, (0, my * n_per), (m_per, n_per))
        out_ref[pl.ds(my * m_per, m_per), :] = own.astype(jnp.float32)

        for j in range(N_DEV):
            @pl.when(my != j)
            def _():
                pltpu.make_async_remote_copy(
                    src_ref=send_buf.at[j],
                    dst_ref=recv_buf.at[j],
                    send_sem=send_sems.at[j],
                    recv_sem=recv_sems.at[j],
                    device_id=(j,),
                    device_id_type=pl.DeviceIdType.MESH,
                ).wait_recv()
                out_ref[j * m_per:(j + 1) * m_per, :] = (
                    recv_buf[j, :, :].astype(jnp.float32))

        for j in range(N_DEV):
            @pl.when(my != j)
            def _():
                pltpu.make_async_remote_copy(
                    src_ref=send_buf.at[j],
                    dst_ref=recv_buf.at[my],
                    send_sem=send_sems.at[j],
                    recv_sem=recv_sems.at[my],
                    device_id=(j,),
                    device_id_type=pl.DeviceIdType.MESH,
                ).wait_send()

    return pl.pallas_call(
        body,
        out_shape=jax.ShapeDtypeStruct((m_total, n_per), jnp.float32),
        in_specs=[pl.BlockSpec(memory_space=pltpu.VMEM),
                  pl.BlockSpec(memory_space=pltpu.VMEM)],
        out_specs=pl.BlockSpec(memory_space=pltpu.VMEM),
        scratch_shapes=[
            pltpu.VMEM((N_DEV, m_per, n_per), jnp.bfloat16),
            pltpu.VMEM((N_DEV, m_per, n_per), jnp.bfloat16),
            pltpu.SemaphoreType.DMA((N_DEV,)),
            pltpu.SemaphoreType.DMA((N_DEV,)),
        ],
        compiler_params=pltpu.CompilerParams(collective_id=0),
    )(x, w_mat)


# baseline (device time: 12351 ns/iter reference)
import jax
import jax.numpy as jnp
from jax import lax
from jax.experimental import pallas as pl
from jax.experimental.pallas import tpu as pltpu

N_DEV = 8


def kernel(x, w_mat):
    m_per, k = x.shape
    n = w_mat.shape[1]
    n_per = n // N_DEV
    m_total = N_DEV * m_per

    def body(x_ref, w_ref, out_ref, send_buf, recv_buf, send_sems, recv_sems):
        my = lax.axis_index("i")

        barrier = pltpu.get_barrier_semaphore()
        for j in range(N_DEV):
            @pl.when(my != j)
            def _():
                pl.semaphore_signal(
                    barrier, inc=1, device_id=(j,),
                    device_id_type=pl.DeviceIdType.MESH)
        pl.semaphore_wait(barrier, N_DEV - 1)

        y = jnp.dot(x_ref[:, :], w_ref[:, :],
                    preferred_element_type=jnp.float32)
        c = 0.7978845608028654
        y = 0.5 * y * (1.0 + jnp.tanh(c * (y + 0.044715 * y * y * y)))
        yb = y.astype(jnp.bfloat16)
        for j in range(N_DEV):
            send_buf[j, :, :] = yb[:, j * n_per:(j + 1) * n_per]

        for j in range(N_DEV):
            @pl.when(my != j)
            def _():
                pltpu.make_async_remote_copy(
                    src_ref=send_buf.at[j],
                    dst_ref=recv_buf.at[my],
                    send_sem=send_sems.at[j],
                    recv_sem=recv_sems.at[my],
                    device_id=(j,),
                    device_id_type=pl.DeviceIdType.MESH,
                ).start()

        own = send_buf[my, :, :]
        out_ref[pl.ds(my * m_per, m_per), :] = own.astype(jnp.float32)

        for j in range(N_DEV):
            @pl.when(my != j)
            def _():
                pltpu.make_async_remote_copy(
                    src_ref=send_buf.at[j],
                    dst_ref=recv_buf.at[j],
                    send_sem=send_sems.at[j],
                    recv_sem=recv_sems.at[j],
                    device_id=(j,),
                    device_id_type=pl.DeviceIdType.MESH,
                ).wait_recv()
                out_ref[j * m_per:(j + 1) * m_per, :] = (
                    recv_buf[j, :, :].astype(jnp.float32))

        for j in range(N_DEV):
            @pl.when(my != j)
            def _():
                pltpu.make_async_remote_copy(
                    src_ref=send_buf.at[j],
                    dst_ref=recv_buf.at[my],
                    send_sem=send_sems.at[j],
                    recv_sem=recv_sems.at[my],
                    device_id=(j,),
                    device_id_type=pl.DeviceIdType.MESH,
                ).wait_send()

    return pl.pallas_call(
        body,
        out_shape=jax.ShapeDtypeStruct((m_total, n_per), jnp.float32),
        in_specs=[pl.BlockSpec(memory_space=pltpu.VMEM),
                  pl.BlockSpec(memory_space=pltpu.VMEM)],
        out_specs=pl.BlockSpec(memory_space=pltpu.VMEM),
        scratch_shapes=[
            pltpu.VMEM((N_DEV, m_per, n_per), jnp.bfloat16),
            pltpu.VMEM((N_DEV, m_per, n_per), jnp.bfloat16),
            pltpu.SemaphoreType.DMA((N_DEV,)),
            pltpu.SemaphoreType.DMA((N_DEV,)),
        ],
        compiler_params=pltpu.CompilerParams(collective_id=0),
    )(x, w_mat)
